# Initial kernel scaffold; baseline (speedup 1.0000x reference)
import functools

import jax
import jax.numpy as jnp
from jax import lax
from jax.experimental import pallas as pl
from jax.experimental.pallas import tpu as pltpu
from jax.experimental.pallas import tpu_sc as plsc

_V = 100000
_B = 16384
_L = 200

_NC = 2
_NS = 16
_NW = _NC * _NS
_GROUPS = _B // 16
_GPW = _GROUPS // _NW
_GPC = 4
_NCHUNK = _GPW // _GPC
_CHW = _GPC * _L * 16


def _diff_body(p_ref, n_ref, o_ref):
    o_ref[...] = p_ref[...] - n_ref[...]


_diff_call = pl.pallas_call(
    _diff_body,
    out_shape=jax.ShapeDtypeStruct((100, _V // 100), jnp.float32),
)


def _sc_body(d_hbm, idx_hbm, out_hbm, d_vmem, idx_a, idx_b, out_vmem,
             sem_d, sem_a, sem_b):
    wid = lax.axis_index("c") * _NS + lax.axis_index("s")
    base = wid * (_NCHUNK * _CHW)

    d_copy = pltpu.async_copy(d_hbm, d_vmem, sem_d)
    bufs = (idx_a, idx_b)
    sems = (sem_a, sem_b)
    copies = [None, None]
    copies[0] = pltpu.async_copy(idx_hbm.at[pl.ds(base, _CHW)], idx_a, sem_a)
    d_copy.wait()

    for c in range(_NCHUNK):
        cur = c % 2
        if c + 1 < _NCHUNK:
            nxt = (c + 1) % 2
            copies[nxt] = pltpu.async_copy(
                idx_hbm.at[pl.ds(base + (c + 1) * _CHW, _CHW)],
                bufs[nxt], sems[nxt])
        copies[cur].wait()
        ibuf = bufs[cur]
        for g in range(_GPC):
            goff = g * (_L * 16)

            def body(l, acc, _goff=goff, _ibuf=ibuf):
                off = pl.multiple_of(_goff + l * 16, 16)
                idxv = _ibuf[pl.ds(off, 16)]
                vals = plsc.load_gather(d_vmem, [idxv])
                return acc + vals

            acc = lax.fori_loop(0, _L, body, jnp.zeros((16,), jnp.float32))
            out_vmem[c * _GPC + g, :] = acc

    pltpu.sync_copy(out_vmem, out_hbm.at[pl.ds(wid * _GPW, _GPW)])


_sc_call = pl.kernel(
    _sc_body,
    out_type=jax.ShapeDtypeStruct((_GROUPS, 16), jnp.float32),
    mesh=plsc.VectorSubcoreMesh(core_axis_name="c", subcore_axis_name="s"),
    scratch_types=[
        pltpu.VMEM((_V,), jnp.float32),
        pltpu.VMEM((_CHW,), jnp.int32),
        pltpu.VMEM((_CHW,), jnp.int32),
        pltpu.VMEM((_GPW, 16), jnp.float32),
        pltpu.SemaphoreType.DMA,
        pltpu.SemaphoreType.DMA,
        pltpu.SemaphoreType.DMA,
    ],
)


def kernel(indices, W_pos, W_neg):
    idx = indices.astype(jnp.int32)
    idx_t = idx.reshape(_GROUPS, 16, _L).transpose(0, 2, 1).reshape(-1)
    d = _diff_call(W_pos.reshape(100, _V // 100),
                   W_neg.reshape(100, _V // 100)).reshape(_V)
    out = _sc_call(d, idx_t)
    return out.reshape(_B)


# speedup vs baseline: 343.7095x; 343.7095x over previous
import functools

import jax
import jax.numpy as jnp
from jax import lax
from jax.experimental import pallas as pl
from jax.experimental.pallas import tpu as pltpu
from jax.experimental.pallas import tpu_sc as plsc

_V = 100000
_B = 16384
_L = 200

_NC = 2
_NS = 16
_NW = _NC * _NS
_GROUPS = _B // 16
_GPW = _GROUPS // _NW
_GPC = 4
_NCHUNK = _GPW // _GPC
_CHW = _GPC * _L * 16


def _diff_body(p_ref, n_ref, o_ref):
    o_ref[...] = p_ref[...] - n_ref[...]


_diff_call = pl.pallas_call(
    _diff_body,
    out_shape=jax.ShapeDtypeStruct((100, _V // 100), jnp.float32),
)


def _sc_body(d_hbm, idx_hbm, out_hbm, d_vmem, idx_a, idx_b, out_vmem,
             sem_d, sem_a, sem_b):
    wid = lax.axis_index("c") * _NS + lax.axis_index("s")
    base = wid * (_NCHUNK * _CHW)

    d_copy = pltpu.async_copy(d_hbm, d_vmem, sem_d)
    bufs = (idx_a, idx_b)
    sems = (sem_a, sem_b)
    copies = [None, None]
    copies[0] = pltpu.async_copy(idx_hbm.at[pl.ds(base, _CHW)], idx_a, sem_a)
    d_copy.wait()

    for c in range(_NCHUNK):
        cur = c % 2
        if c + 1 < _NCHUNK:
            nxt = (c + 1) % 2
            copies[nxt] = pltpu.async_copy(
                idx_hbm.at[pl.ds(base + (c + 1) * _CHW, _CHW)],
                bufs[nxt], sems[nxt])
        copies[cur].wait()
        ibuf = bufs[cur]
        for g in range(_GPC):
            goff = g * (_L * 16)

            def body(l, acc, _goff=goff, _ibuf=ibuf):
                off = pl.multiple_of(_goff + l * 16, 16)
                idxv = _ibuf[pl.ds(off, 16)]
                vals = plsc.load_gather(d_vmem, [idxv])
                return acc + vals

            acc = lax.fori_loop(0, _L, body, jnp.zeros((16,), jnp.float32))
            out_vmem[c * _GPC + g, :] = acc

    pltpu.sync_copy(out_vmem, out_hbm.at[pl.ds(wid * _GPW, _GPW)])


_sc_call = pl.kernel(
    _sc_body,
    out_type=jax.ShapeDtypeStruct((_GROUPS, 16), jnp.float32),
    mesh=plsc.VectorSubcoreMesh(core_axis_name="c", subcore_axis_name="s"),
    compiler_params=pltpu.CompilerParams(needs_layout_passes=False),
    scratch_types=[
        pltpu.VMEM((_V,), jnp.float32),
        pltpu.VMEM((_CHW,), jnp.int32),
        pltpu.VMEM((_CHW,), jnp.int32),
        pltpu.VMEM((_GPW, 16), jnp.float32),
        pltpu.SemaphoreType.DMA,
        pltpu.SemaphoreType.DMA,
        pltpu.SemaphoreType.DMA,
    ],
)


def kernel(indices, W_pos, W_neg):
    idx = indices.astype(jnp.int32)
    idx_t = idx.reshape(_GROUPS, 16, _L).transpose(0, 2, 1).reshape(-1)
    d = _diff_call(W_pos.reshape(100, _V // 100),
                   W_neg.reshape(100, _V // 100)).reshape(_V)
    out = _sc_call(d, idx_t)
    return out.reshape(_B)


# speedup vs baseline: 682.1956x; 1.9848x over previous
import functools

import jax
import jax.numpy as jnp
from jax import lax
from jax.experimental import pallas as pl
from jax.experimental.pallas import tpu as pltpu
from jax.experimental.pallas import tpu_sc as plsc

_V = 100000
_B = 16384
_L = 200

_NC = 2
_NS = 16
_NW = _NC * _NS
_GROUPS = _B // 16
_GPW = _GROUPS // _NW
_GPC = 4
_NCHUNK = _GPW // _GPC
_CHW = _GPC * _L * 16


def _diff_body(p_ref, n_ref, o_ref):
    o_ref[...] = p_ref[...] - n_ref[...]


_diff_call = pl.pallas_call(
    _diff_body,
    out_shape=jax.ShapeDtypeStruct((100, _V // 100), jnp.float32),
)


_UNROLL = 8


def _sc_body(d_hbm, idx_hbm, out_hbm, d_vmem, idx_a, idx_b, out_vmem,
             sem_d, sem_a, sem_b):
    wid = lax.axis_index("c") * _NS + lax.axis_index("s")
    base = wid * (_NCHUNK * _CHW)

    d_copy = pltpu.async_copy(d_hbm, d_vmem, sem_d)
    bufs = (idx_a, idx_b)
    sems = (sem_a, sem_b)
    copies = [None, None]
    copies[0] = pltpu.async_copy(idx_hbm.at[pl.ds(base, _CHW)], idx_a, sem_a)
    d_copy.wait()

    lane = lax.iota(jnp.int32, 16)
    row_base0 = lane * _L

    for c in range(_NCHUNK):
        cur = c % 2
        if c + 1 < _NCHUNK:
            nxt = (c + 1) % 2
            copies[nxt] = pltpu.async_copy(
                idx_hbm.at[pl.ds(base + (c + 1) * _CHW, _CHW)],
                bufs[nxt], sems[nxt])
        copies[cur].wait()
        ibuf = bufs[cur]
        for g in range(_GPC):
            row_base = row_base0 + (g * 16 * _L)

            def body(_, carry, _row_base=row_base, _ibuf=ibuf):
                rel, a0, a1 = carry
                for u in range(_UNROLL):
                    idxv = plsc.load_gather(_ibuf, [_row_base + rel])
                    vals = plsc.load_gather(d_vmem, [idxv])
                    if u % 2 == 0:
                        a0 = a0 + vals
                    else:
                        a1 = a1 + vals
                    rel = rel + 1
                    rel = jnp.where(rel == _L, 0, rel)
                return rel, a0, a1

            zero = jnp.zeros((16,), jnp.float32)
            _, a0, a1 = lax.fori_loop(0, _L // _UNROLL, body,
                                      (lane, zero, zero))
            out_vmem[c * _GPC + g, :] = a0 + a1

    pltpu.sync_copy(out_vmem, out_hbm.at[pl.ds(wid * _GPW, _GPW)])


_sc_call = pl.kernel(
    _sc_body,
    out_type=jax.ShapeDtypeStruct((_GROUPS, 16), jnp.float32),
    mesh=plsc.VectorSubcoreMesh(core_axis_name="c", subcore_axis_name="s"),
    compiler_params=pltpu.CompilerParams(needs_layout_passes=False),
    scratch_types=[
        pltpu.VMEM((_V,), jnp.float32),
        pltpu.VMEM((_CHW,), jnp.int32),
        pltpu.VMEM((_CHW,), jnp.int32),
        pltpu.VMEM((_GPW, 16), jnp.float32),
        pltpu.SemaphoreType.DMA,
        pltpu.SemaphoreType.DMA,
        pltpu.SemaphoreType.DMA,
    ],
)


def kernel(indices, W_pos, W_neg):
    idx_flat = indices.astype(jnp.int32).reshape(-1)
    d = _diff_call(W_pos.reshape(100, _V // 100),
                   W_neg.reshape(100, _V // 100)).reshape(_V)
    out = _sc_call(d, idx_flat)
    return out.reshape(_B)


# speedup vs baseline: 821.9073x; 1.2048x over previous
import functools

import jax
import jax.numpy as jnp
from jax import lax
from jax.experimental import pallas as pl
from jax.experimental.pallas import tpu as pltpu
from jax.experimental.pallas import tpu_sc as plsc

_V = 100000
_B = 16384
_L = 200

_NC = 2
_NS = 16
_NW = _NC * _NS
_GROUPS = _B // 16
_GPW = _GROUPS // _NW
_GPC = 2
_NCHUNK = _GPW // _GPC
_CHW = _GPC * _L * 16


def _diff_body(p_ref, n_ref, o_ref):
    o_ref[...] = p_ref[...] - n_ref[...]


_diff_call = pl.pallas_call(
    _diff_body,
    out_shape=jax.ShapeDtypeStruct((100, _V // 100), jnp.float32),
)


_UNROLL = 8


def _sc_body(d_hbm, idx_hbm, out_hbm, d_vmem, idx_a, idx_b, out_vmem,
             sem_d, sem_a, sem_b):
    wid = lax.axis_index("c") * _NS + lax.axis_index("s")

    d_copy = pltpu.async_copy(d_hbm, d_vmem, sem_d)
    bufs = (idx_a, idx_b)
    sems = (sem_a, sem_b)
    copies = [None, None]
    rows_per_chunk = _GPC * 16
    base_row = wid * _GPW * 16
    copies[0] = pltpu.async_copy(
        idx_hbm.at[pl.ds(base_row, rows_per_chunk)], idx_a, sem_a)
    d_copy.wait()

    lane = lax.iota(jnp.int32, 16)

    for c in range(_NCHUNK):
        cur = c % 2
        if c + 1 < _NCHUNK:
            nxt = (c + 1) % 2
            copies[nxt] = pltpu.async_copy(
                idx_hbm.at[pl.ds(base_row + (c + 1) * rows_per_chunk,
                                 rows_per_chunk)],
                bufs[nxt], sems[nxt])
        copies[cur].wait()
        ibuf = bufs[cur]
        for g in range(_GPC):
            rowv = lane + (g * 16)

            def body(_, carry, _rowv=rowv, _ibuf=ibuf):
                rel0, a0, a1 = carry
                for u in range(_UNROLL):
                    relu = rel0 + u
                    relu = jnp.where(relu >= _L, relu - _L, relu)
                    idxv = plsc.load_gather(_ibuf, [_rowv, relu])
                    vals = plsc.load_gather(d_vmem, [idxv])
                    if u % 2 == 0:
                        a0 = a0 + vals
                    else:
                        a1 = a1 + vals
                rel0 = rel0 + _UNROLL
                rel0 = jnp.where(rel0 >= _L, rel0 - _L, rel0)
                return rel0, a0, a1

            zero = jnp.zeros((16,), jnp.float32)
            _, a0, a1 = lax.fori_loop(0, _L // _UNROLL, body,
                                      (lane, zero, zero))
            out_vmem[c * _GPC + g, :] = a0 + a1

    pltpu.sync_copy(out_vmem, out_hbm.at[pl.ds(wid * _GPW, _GPW)])


_sc_call = pl.kernel(
    _sc_body,
    out_type=jax.ShapeDtypeStruct((_GROUPS, 16), jnp.float32),
    mesh=plsc.VectorSubcoreMesh(core_axis_name="c", subcore_axis_name="s"),
    compiler_params=pltpu.CompilerParams(needs_layout_passes=False),
    scratch_types=[
        pltpu.VMEM((_V,), jnp.float32),
        pltpu.VMEM((_GPC * 16, _L), jnp.int32),
        pltpu.VMEM((_GPC * 16, _L), jnp.int32),
        pltpu.VMEM((_GPW, 16), jnp.float32),
        pltpu.SemaphoreType.DMA,
        pltpu.SemaphoreType.DMA,
        pltpu.SemaphoreType.DMA,
    ],
)


def kernel(indices, W_pos, W_neg):
    d = _diff_call(W_pos.reshape(100, _V // 100),
                   W_neg.reshape(100, _V // 100)).reshape(_V)
    out = _sc_call(d, indices.astype(jnp.int32))
    return out.reshape(_B)


# speedup vs baseline: 846.1181x; 1.0295x over previous
import functools

import jax
import jax.numpy as jnp
from jax import lax
from jax.experimental import pallas as pl
from jax.experimental.pallas import tpu as pltpu
from jax.experimental.pallas import tpu_sc as plsc

_V = 100000
_B = 16384
_L = 200

_NC = 2
_NS = 16
_NW = _NC * _NS
_GROUPS = _B // 16
_GPW = _GROUPS // _NW
_GPC = 2
_NCHUNK = _GPW // _GPC
_CHW = _GPC * _L * 16


def _diff_body(p_ref, n_ref, o_ref):
    o_ref[...] = p_ref[...] - n_ref[...]


_diff_call = pl.pallas_call(
    _diff_body,
    out_shape=jax.ShapeDtypeStruct((100, _V // 100), jnp.float32),
)


_UNROLL = 8


def _sc_body(d_hbm, idx_hbm, out_hbm, d_vmem, idx_a, idx_b, out_vmem,
             sem_d, sem_a, sem_b):
    wid = lax.axis_index("c") * _NS + lax.axis_index("s")

    d_copy = pltpu.async_copy(d_hbm, d_vmem, sem_d)
    bufs = (idx_a, idx_b)
    sems = (sem_a, sem_b)
    copies = [None, None]
    rows_per_chunk = _GPC * 16
    base_row = wid * _GPW * 16
    copies[0] = pltpu.async_copy(
        idx_hbm.at[pl.ds(base_row, rows_per_chunk)], idx_a, sem_a)
    d_copy.wait()

    lane = lax.iota(jnp.int32, 16)

    for c in range(_NCHUNK):
        cur = c % 2
        if c + 1 < _NCHUNK:
            nxt = (c + 1) % 2
            copies[nxt] = pltpu.async_copy(
                idx_hbm.at[pl.ds(base_row + (c + 1) * rows_per_chunk,
                                 rows_per_chunk)],
                bufs[nxt], sems[nxt])
        copies[cur].wait()
        ibuf = bufs[cur]
        for g in range(_GPC):
            rowv = lane + (g * 16)

            def body(_, carry, _rowv=rowv, _ibuf=ibuf):
                rel0, a0, a1 = carry
                for u in range(_UNROLL):
                    relu = rel0 + u
                    relu = jnp.where(relu >= _L, relu - _L, relu)
                    idxv = plsc.load_gather(_ibuf, [_rowv, relu])
                    vals = plsc.load_gather(d_vmem, [idxv])
                    if u % 2 == 0:
                        a0 = a0 + vals
                    else:
                        a1 = a1 + vals
                rel0 = rel0 + _UNROLL
                rel0 = jnp.where(rel0 >= _L, rel0 - _L, rel0)
                return rel0, a0, a1

            zero = jnp.zeros((16,), jnp.float32)
            _, a0, a1 = lax.fori_loop(0, _L // _UNROLL, body,
                                      (lane, zero, zero))
            out_vmem[pl.ds((c * _GPC + g) * 16, 16)] = a0 + a1

    pltpu.sync_copy(out_vmem, out_hbm.at[pl.ds(wid * _GPW * 16, _GPW * 16)])


_sc_call = pl.kernel(
    _sc_body,
    out_type=jax.ShapeDtypeStruct((_B,), jnp.float32),
    mesh=plsc.VectorSubcoreMesh(core_axis_name="c", subcore_axis_name="s"),
    compiler_params=pltpu.CompilerParams(needs_layout_passes=False,
                                         use_tc_tiling_on_sc=True),
    scratch_types=[
        pltpu.VMEM((_V,), jnp.float32),
        pltpu.VMEM((_GPC * 16, _L), jnp.int32),
        pltpu.VMEM((_GPC * 16, _L), jnp.int32),
        pltpu.VMEM((_GPW * 16,), jnp.float32),
        pltpu.SemaphoreType.DMA,
        pltpu.SemaphoreType.DMA,
        pltpu.SemaphoreType.DMA,
    ],
)


def kernel(indices, W_pos, W_neg):
    d = _diff_call(W_pos.reshape(100, _V // 100),
                   W_neg.reshape(100, _V // 100)).reshape(_V)
    return _sc_call(d, indices.astype(jnp.int32))


# speedup vs baseline: 862.3211x; 1.0191x over previous
import functools

import jax
import jax.numpy as jnp
from jax import lax
from jax.experimental import pallas as pl
from jax.experimental.pallas import tpu as pltpu
from jax.experimental.pallas import tpu_sc as plsc

_V = 100000
_B = 16384
_L = 200

_NC = 2
_NS = 16
_NW = _NC * _NS
_GROUPS = _B // 16
_GPW = _GROUPS // _NW
_GPC = 2
_NCHUNK = _GPW // _GPC
_CHW = _GPC * _L * 16


def _diff_body(p_ref, n_ref, o_ref):
    o_ref[...] = p_ref[...] - n_ref[...]


_diff_call = pl.pallas_call(
    _diff_body,
    out_shape=jax.ShapeDtypeStruct((_V,), jnp.float32),
)


_UNROLL = 16


def _sc_body(d_hbm, idx_hbm, out_hbm, d_vmem, idx_a, idx_b, out_vmem,
             sem_d, sem_a, sem_b):
    wid = lax.axis_index("c") * _NS + lax.axis_index("s")

    d_copy = pltpu.async_copy(d_hbm, d_vmem, sem_d)
    bufs = (idx_a, idx_b)
    sems = (sem_a, sem_b)
    copies = [None, None]
    rows_per_chunk = _GPC * 16
    base_row = wid * _GPW * 16
    copies[0] = pltpu.async_copy(
        idx_hbm.at[pl.ds(base_row, rows_per_chunk)], idx_a, sem_a)
    d_copy.wait()

    lane = lax.iota(jnp.int32, 16)

    for c in range(_NCHUNK):
        cur = c % 2
        if c + 1 < _NCHUNK:
            nxt = (c + 1) % 2
            copies[nxt] = pltpu.async_copy(
                idx_hbm.at[pl.ds(base_row + (c + 1) * rows_per_chunk,
                                 rows_per_chunk)],
                bufs[nxt], sems[nxt])
        copies[cur].wait()
        ibuf = bufs[cur]
        for g in range(_GPC):
            rowv = lane + (g * 16)

            def body(_, carry, _rowv=rowv, _ibuf=ibuf):
                rel0, a0, a1, a2, a3 = carry
                accs = [a0, a1, a2, a3]
                for u in range(_UNROLL):
                    relu = rel0 + u
                    relu = jnp.where(relu >= _L, relu - _L, relu)
                    idxv = plsc.load_gather(_ibuf, [_rowv, relu])
                    vals = plsc.load_gather(d_vmem, [idxv])
                    accs[u % 4] = accs[u % 4] + vals
                rel0 = rel0 + _UNROLL
                rel0 = jnp.where(rel0 >= _L, rel0 - _L, rel0)
                return (rel0, *accs)

            zero = jnp.zeros((16,), jnp.float32)
            _, a0, a1, a2, a3 = lax.fori_loop(
                0, _L // _UNROLL, body, (lane, zero, zero, zero, zero))
            out_vmem[pl.ds((c * _GPC + g) * 16, 16)] = (a0 + a1) + (a2 + a3)

    pltpu.sync_copy(out_vmem, out_hbm.at[pl.ds(wid * _GPW * 16, _GPW * 16)])


_sc_call = pl.kernel(
    _sc_body,
    out_type=jax.ShapeDtypeStruct((_B,), jnp.float32),
    mesh=plsc.VectorSubcoreMesh(core_axis_name="c", subcore_axis_name="s"),
    compiler_params=pltpu.CompilerParams(needs_layout_passes=False,
                                         use_tc_tiling_on_sc=True),
    scratch_types=[
        pltpu.VMEM((_V,), jnp.float32),
        pltpu.VMEM((_GPC * 16, _L), jnp.int32),
        pltpu.VMEM((_GPC * 16, _L), jnp.int32),
        pltpu.VMEM((_GPW * 16,), jnp.float32),
        pltpu.SemaphoreType.DMA,
        pltpu.SemaphoreType.DMA,
        pltpu.SemaphoreType.DMA,
    ],
)


def kernel(indices, W_pos, W_neg):
    d = _diff_call(W_pos.reshape(_V), W_neg.reshape(_V))
    return _sc_call(d, indices.astype(jnp.int32))
